# baseline (speedup 1.0000x reference)
import functools

import jax
import jax.numpy as jnp
from jax import lax
from jax.experimental import pallas as pl
from jax.experimental.pallas import tpu as pltpu
from jax.experimental.pallas import tpu_sc as plsc

_TILE_M = 1024
_NC, _NS, _L = 2, 16, 16
_NW = _NC * _NS


def _logits_body(x_ref, wt_ref, b_ref, out_ref):
    out_ref[...] = (
        jnp.dot(x_ref[...], wt_ref[...], preferred_element_type=jnp.float32)
        + b_ref[...]
    )


def _tc_logits(x, wt, b2, row0, rows):
    M, D = x.shape
    E = wt.shape[1]
    tile0 = row0 // _TILE_M
    return pl.pallas_call(
        _logits_body,
        grid=(rows // _TILE_M,),
        in_specs=[
            pl.BlockSpec((_TILE_M, D), lambda i: (i + tile0, 0)),
            pl.BlockSpec((D, E), lambda i: (0, 0)),
            pl.BlockSpec((1, E), lambda i: (0, 0)),
        ],
        out_specs=pl.BlockSpec((_TILE_M, E), lambda i: (i, 0)),
        out_shape=jax.ShapeDtypeStruct((rows, E), jnp.float32),
        compiler_params=pltpu.CompilerParams(
            dimension_semantics=("arbitrary",)),
    )(x, wt, b2)


def _sc_route(logits, cond16, M):
    rows_per_w = M // _NW
    chunk = min(128, rows_per_w)
    n_chunks = rows_per_w // chunk
    mesh = plsc.VectorSubcoreMesh(
        core_axis_name="c", subcore_axis_name="s",
        num_cores=_NC, num_subcores=_NS)

    @functools.partial(
        pl.kernel,
        out_type=jax.ShapeDtypeStruct((M, 64), jnp.float32),
        mesh=mesh,
        scratch_types=[
            pltpu.VMEM((chunk, 64), jnp.float32),
            pltpu.VMEM((chunk, 64), jnp.float32),
            pltpu.VMEM((chunk, 64), jnp.float32),
            pltpu.VMEM((chunk, 64), jnp.float32),
            pltpu.VMEM((_L,), jnp.int32),
            pltpu.SemaphoreType.DMA,
            pltpu.SemaphoreType.DMA,
            pltpu.SemaphoreType.DMA,
            pltpu.SemaphoreType.DMA,
        ],
        compiler_params=pltpu.CompilerParams(needs_layout_passes=False),
    )
    def route(logits_hbm, cond_hbm, out_hbm,
              in_a, in_b, out_a, out_b, cond_v, si_a, si_b, so_a, so_b):
        wid = lax.axis_index("s") * _NC + lax.axis_index("c")
        row0 = wid * rows_per_w
        first_in = pltpu.make_async_copy(
            logits_hbm.at[pl.ds(row0, chunk)], in_a, si_a)
        first_in.start()
        pltpu.sync_copy(cond_hbm, cond_v)
        cond = cond_v[...] != 0
        iota = lax.iota(jnp.int32, _L)
        lo8 = iota < 8
        seven = jnp.full((_L,), 7, jnp.int32)
        zero = jnp.zeros((_L,), jnp.int32)
        ins = (in_a, in_b)
        outs = (out_a, out_b)
        sis = (si_a, si_b)
        sos = (so_a, so_b)

        shuf_dn = lax.GatherDimensionNumbers(
            offset_dims=(), collapsed_slice_dims=(0,), start_index_map=(0,))

        def shuf(v, idx):
            return lax.gather(
                v, idx[:, None], shuf_dn, (1,),
                mode=lax.GatherScatterMode.PROMISE_IN_BOUNDS)

        def dsort(v):
            return plsc.sort_key_val(v, v, descending=True)[0]

        def make_row_body(in_v, out_v):
          def row_body(r):
            a0 = in_v[r, pl.ds(0, _L)]
            a1 = in_v[r, pl.ds(16, _L)]
            a2 = in_v[r, pl.ds(32, _L)]
            a3 = in_v[r, pl.ds(48, _L)]
            s0 = dsort(a0)
            s1 = dsort(a1)
            s2 = dsort(a2)
            s3 = dsort(a3)
            t01 = dsort(jnp.where(lo8, s0, lax.rev(s1, (0,))))
            t23 = dsort(jnp.where(lo8, s2, lax.rev(s3, (0,))))
            f = dsort(jnp.where(lo8, t01, lax.rev(t23, (0,))))
            m1v = shuf(f, zero)
            threshv = shuf(f, seven)
            g0 = jnp.where(cond | (a0 >= threshv), jnp.exp(a0 - m1v), 0.0)
            g1 = jnp.where(cond | (a1 >= threshv), jnp.exp(a1 - m1v), 0.0)
            g2 = jnp.where(cond | (a2 >= threshv), jnp.exp(a2 - m1v), 0.0)
            g3 = jnp.where(cond | (a3 >= threshv), jnp.exp(a3 - m1v), 0.0)
            sv = (g0 + g1) + (g2 + g3)
            for sh in (8, 4, 2, 1):
                sv = sv + shuf(sv, iota ^ sh)
            rcp = 1.0 / sv
            out_v[r, pl.ds(0, _L)] = g0 * rcp
            out_v[r, pl.ds(16, _L)] = g1 * rcp
            out_v[r, pl.ds(32, _L)] = g2 * rcp
            out_v[r, pl.ds(48, _L)] = g3 * rcp
          return row_body

        def start_in(ci):
            r0 = row0 + ci * chunk
            return pltpu.make_async_copy(
                logits_hbm.at[pl.ds(r0, chunk)], ins[ci % 2], sis[ci % 2])

        def start_out(ci):
            r0 = row0 + ci * chunk
            return pltpu.make_async_copy(
                outs[ci % 2], out_hbm.at[pl.ds(r0, chunk)], sos[ci % 2])

        h_in = [first_in]
        h_out = []
        for ci in range(n_chunks):
            if ci + 1 < n_chunks:
                nxt = start_in(ci + 1)
                nxt.start()
                h_in.append(nxt)
            h_in[ci].wait()
            if ci >= 2:
                h_out[ci - 2].wait()
            plsc.parallel_loop(0, chunk, 1, unroll=3)(
                make_row_body(ins[ci % 2], outs[ci % 2]))
            ocp = start_out(ci)
            ocp.start()
            h_out.append(ocp)
        for h in h_out[-2:]:
            h.wait()

    return route(logits, cond16)


def kernel(x, W, b, k, training):
    del training
    M, D = x.shape
    E = W.shape[0]
    cond16 = jnp.full(
        (_L,), (jnp.asarray(k, jnp.int32) >= E).astype(jnp.int32))
    logits = _tc_logits(x, W.T, b.reshape(1, E), 0, M)
    return _sc_route(logits, cond16, M)


# speedup vs baseline: 1.0123x; 1.0123x over previous
import functools

import jax
import jax.numpy as jnp
from jax import lax
from jax.experimental import pallas as pl
from jax.experimental.pallas import tpu as pltpu
from jax.experimental.pallas import tpu_sc as plsc

_TILE_M = 1024
_NC, _NS, _L = 2, 16, 16
_NW = _NC * _NS


def _logits_body(x_ref, wt_ref, b_ref, out_ref):
    out_ref[...] = (
        jnp.dot(x_ref[...], wt_ref[...], preferred_element_type=jnp.float32)
        + b_ref[...]
    )


def _tc_logits(x, wt, b2, row0, rows):
    M, D = x.shape
    E = wt.shape[1]
    tile0 = row0 // _TILE_M
    return pl.pallas_call(
        _logits_body,
        grid=(rows // _TILE_M,),
        in_specs=[
            pl.BlockSpec((_TILE_M, D), lambda i: (i + tile0, 0)),
            pl.BlockSpec((D, E), lambda i: (0, 0)),
            pl.BlockSpec((1, E), lambda i: (0, 0)),
        ],
        out_specs=pl.BlockSpec((_TILE_M, E), lambda i: (i, 0)),
        out_shape=jax.ShapeDtypeStruct((rows, E), jnp.float32),
        compiler_params=pltpu.CompilerParams(
            dimension_semantics=("arbitrary",)),
    )(x, wt, b2)


def _sc_route(logits, cond16, M):
    rows_per_w = M // _NW
    chunk = min(128, rows_per_w)
    n_chunks = rows_per_w // chunk
    mesh = plsc.VectorSubcoreMesh(
        core_axis_name="c", subcore_axis_name="s",
        num_cores=_NC, num_subcores=_NS)

    @functools.partial(
        pl.kernel,
        out_type=jax.ShapeDtypeStruct((M, 64), jnp.float32),
        mesh=mesh,
        scratch_types=[
            pltpu.VMEM((chunk, 64), jnp.float32),
            pltpu.VMEM((chunk, 64), jnp.float32),
            pltpu.VMEM((chunk, 64), jnp.float32),
            pltpu.VMEM((chunk, 64), jnp.float32),
            pltpu.VMEM((_L,), jnp.int32),
            pltpu.SemaphoreType.DMA,
            pltpu.SemaphoreType.DMA,
            pltpu.SemaphoreType.DMA,
            pltpu.SemaphoreType.DMA,
        ],
        compiler_params=pltpu.CompilerParams(needs_layout_passes=False),
    )
    def route(logits_hbm, cond_hbm, out_hbm,
              in_a, in_b, out_a, out_b, cond_v, si_a, si_b, so_a, so_b):
        wid = lax.axis_index("s") * _NC + lax.axis_index("c")
        row0 = wid * rows_per_w
        first_in = pltpu.make_async_copy(
            logits_hbm.at[pl.ds(row0, chunk)], in_a, si_a)
        first_in.start()
        pltpu.sync_copy(cond_hbm, cond_v)
        cond = cond_v[...] != 0
        iota = lax.iota(jnp.int32, _L)
        lo8 = iota < 8
        seven = jnp.full((_L,), 7, jnp.int32)
        zero = jnp.zeros((_L,), jnp.int32)
        ins = (in_a, in_b)
        outs = (out_a, out_b)
        sis = (si_a, si_b)
        sos = (so_a, so_b)

        shuf_dn = lax.GatherDimensionNumbers(
            offset_dims=(), collapsed_slice_dims=(0,), start_index_map=(0,))

        def shuf(v, idx):
            return lax.gather(
                v, idx[:, None], shuf_dn, (1,),
                mode=lax.GatherScatterMode.PROMISE_IN_BOUNDS)

        def dsort(v):
            return plsc.sort_key_val(v, v, descending=True)[0]

        def make_row_body(in_v, out_v):
          def row_body(r):
            a0 = in_v[r, pl.ds(0, _L)]
            a1 = in_v[r, pl.ds(16, _L)]
            a2 = in_v[r, pl.ds(32, _L)]
            a3 = in_v[r, pl.ds(48, _L)]
            s0 = dsort(a0)
            s1 = dsort(a1)
            s2 = dsort(a2)
            s3 = dsort(a3)
            t01 = dsort(jnp.where(lo8, s0, lax.rev(s1, (0,))))
            t23 = dsort(jnp.where(lo8, s2, lax.rev(s3, (0,))))
            f = dsort(jnp.where(lo8, t01, lax.rev(t23, (0,))))
            m1v = shuf(f, zero)
            threshv = shuf(f, seven)
            g0 = jnp.where(cond | (a0 >= threshv), jnp.exp(a0 - m1v), 0.0)
            g1 = jnp.where(cond | (a1 >= threshv), jnp.exp(a1 - m1v), 0.0)
            g2 = jnp.where(cond | (a2 >= threshv), jnp.exp(a2 - m1v), 0.0)
            g3 = jnp.where(cond | (a3 >= threshv), jnp.exp(a3 - m1v), 0.0)
            sv = (g0 + g1) + (g2 + g3)
            for sh in (8, 4, 2, 1):
                sv = sv + shuf(sv, iota ^ sh)
            rcp = 1.0 / sv
            out_v[r, pl.ds(0, _L)] = g0 * rcp
            out_v[r, pl.ds(16, _L)] = g1 * rcp
            out_v[r, pl.ds(32, _L)] = g2 * rcp
            out_v[r, pl.ds(48, _L)] = g3 * rcp
          return row_body

        def start_in(ci):
            r0 = row0 + ci * chunk
            return pltpu.make_async_copy(
                logits_hbm.at[pl.ds(r0, chunk)], ins[ci % 2], sis[ci % 2])

        def start_out(ci):
            r0 = row0 + ci * chunk
            return pltpu.make_async_copy(
                outs[ci % 2], out_hbm.at[pl.ds(r0, chunk)], sos[ci % 2])

        h_in = [first_in]
        h_out = []
        for ci in range(n_chunks):
            if ci + 1 < n_chunks:
                nxt = start_in(ci + 1)
                nxt.start()
                h_in.append(nxt)
            h_in[ci].wait()
            if ci >= 2:
                h_out[ci - 2].wait()
            plsc.parallel_loop(0, chunk, 1, unroll=2)(
                make_row_body(ins[ci % 2], outs[ci % 2]))
            ocp = start_out(ci)
            ocp.start()
            h_out.append(ocp)
        for h in h_out[-2:]:
            h.wait()

    return route(logits, cond16)


def kernel(x, W, b, k, training):
    del training
    M, D = x.shape
    E = W.shape[0]
    cond16 = jnp.full(
        (_L,), (jnp.asarray(k, jnp.int32) >= E).astype(jnp.int32))
    logits = _tc_logits(x, W.T, b.reshape(1, E), 0, M)
    return _sc_route(logits, cond16, M)
